# baseline (speedup 1.0000x reference)
import functools

import jax
import jax.numpy as jnp
from jax import lax
from jax.experimental import pallas as pl
from jax.experimental.pallas import tpu as pltpu
from jax.experimental.pallas import tpu_sc as plsc

Q = 1024
K = 100000
D = 128
EPS = 1e-8
KBLK = 2048

NT = 16
NC = 2
C = 6272
KP = NT * C
NV = C // 16
U = 4
NVU = NV // U
CU = NVU * 16
RADIX = 2048
SHIFTS = (0, 11, 22)
ROWS_PER_CORE = Q // NC
TAIL = K - (NT - 1) * C

_SIGN_BITS = 0x80000000


def _scores_body(q_ref, k_ref, qn_ref, kn_ref, out_ref):
    dot = jax.lax.dot_general(q_ref[...], k_ref[...], (((1,), (1,)), ((), ())))
    denom = jnp.maximum(qn_ref[...] * kn_ref[0:1, :], EPS)
    out_ref[...] = dot / denom


def _scores(queries, keys):
    qn = jnp.linalg.norm(queries, axis=1, keepdims=True)
    kn = jnp.linalg.norm(keys, axis=1, keepdims=True)
    kn2d = jnp.broadcast_to(kn.T, (8, K))
    return pl.pallas_call(
        _scores_body,
        grid=(pl.cdiv(K, KBLK),),
        in_specs=[
            pl.BlockSpec((Q, D), lambda i: (0, 0)),
            pl.BlockSpec((KBLK, D), lambda i: (i, 0)),
            pl.BlockSpec((Q, 1), lambda i: (0, 0)),
            pl.BlockSpec((8, KBLK), lambda i: (0, i)),
        ],
        out_specs=pl.BlockSpec((Q, KBLK), lambda i: (0, i)),
        out_shape=jax.ShapeDtypeStruct((Q, KP), jnp.float32),
    )(queries, keys, qn, kn2d)


_mesh = plsc.VectorSubcoreMesh(core_axis_name="c", subcore_axis_name="s")


@functools.partial(
    pl.kernel,
    out_type=[
        jax.ShapeDtypeStruct((Q * K,), jnp.float32),
        jax.ShapeDtypeStruct((Q * K,), jnp.int32),
    ],
    mesh=_mesh,
    compiler_params=pltpu.CompilerParams(needs_layout_passes=False),
    scratch_types=[
        pltpu.VMEM_SHARED((KP,), jnp.int32),
        pltpu.VMEM_SHARED((KP,), jnp.int32),
        pltpu.VMEM_SHARED((KP,), jnp.int32),
        pltpu.VMEM_SHARED((KP,), jnp.int32),
        pltpu.VMEM_SHARED((NT * RADIX,), jnp.int32),
        pltpu.VMEM_SHARED((NT * RADIX,), jnp.int32),
        pltpu.VMEM_SHARED((NT * 16,), jnp.int32),
        pltpu.VMEM((C,), jnp.int32),
        pltpu.VMEM((C,), jnp.int32),
        pltpu.VMEM((C,), jnp.int32),
        pltpu.VMEM((C,), jnp.float32),
        [pltpu.VMEM((RADIX,), jnp.int32) for _ in range(U)],
        [pltpu.VMEM((RADIX,), jnp.int32) for _ in range(U)],
        pltpu.VMEM((RADIX,), jnp.int32),
        pltpu.VMEM((RADIX,), jnp.int32),
        pltpu.VMEM((NT * 128,), jnp.int32),
        pltpu.VMEM((NT * 128,), jnp.int32),
        pltpu.VMEM((128,), jnp.int32),
        pltpu.VMEM((NT * 16,), jnp.int32),
        pltpu.VMEM((16,), jnp.int32),
        pltpu.SemaphoreType.DMA,
        pltpu.SemaphoreType.DMA,
    ],
)
def _sort_kernel(scores_hbm, sc_out, ord_out, KA, VA, KB, VB, HG, CURG, TS2,
                 keych, valch, posbuf, scorech, hu, cu, hist, cur, A, CURbuf,
                 lexcl, TSl, TSbuf, sem1, sem2):
    c = lax.axis_index("c")
    s = lax.axis_index("s")
    lanes = lax.iota(jnp.int32, 16)
    zeros16 = jnp.zeros((16,), jnp.int32)

    def do_pass(shift, dst_k, dst_v):
        def z(i, _):
            for u in range(U):
                hu[u][pl.ds(i * 16, 16)] = zeros16
            return 0
        lax.fori_loop(0, RADIX // 16, z, 0)

        def digit(off):
            kk = plsc.bitcast(keych[pl.ds(off, 16)], jnp.uint32)
            return ((kk >> shift) & jnp.uint32(RADIX - 1)).astype(jnp.int32)

        def hsweep(i, _):
            for u in range(U):
                d = digit(u * CU + i * 16)
                cnt, lastm = plsc.scan_count(d)
                old = plsc.load_gather(hu[u], [d])
                plsc.store_scatter(hu[u], [d], old + cnt, mask=lastm)
            return 0
        lax.fori_loop(0, NVU, hsweep, 0)

        def comb(g, _):
            hist[pl.ds(g * 16, 16)] = (
                hu[0][pl.ds(g * 16, 16)] + hu[1][pl.ds(g * 16, 16)]
                + hu[2][pl.ds(g * 16, 16)] + hu[3][pl.ds(g * 16, 16)])
            return 0
        lax.fori_loop(0, RADIX // 16, comb, 0)

        pltpu.sync_copy(hist, HG.at[pl.ds(s * RADIX, RADIX)])
        plsc.subcore_barrier()

        def rd(u, _):
            pltpu.sync_copy(HG.at[pl.ds(u * RADIX + s * 128, 128)],
                            A.at[pl.ds(u * 128, 128)])
            return 0
        lax.fori_loop(0, NT, rd, 0)

        def grp(g, _):
            def inner(u, acc):
                CURbuf[pl.ds(u * 128 + g * 16, 16)] = acc
                return acc + A[pl.ds(u * 128 + g * 16, 16)]
            tot = lax.fori_loop(0, NT, inner, zeros16)
            lexcl[pl.ds(g * 16, 16)] = tot
            return 0
        lax.fori_loop(0, 8, grp, 0)

        def lscan(g, carry):
            v = lexcl[pl.ds(g * 16, 16)]
            inc = plsc.cumsum(v)
            lexcl[pl.ds(g * 16, 16)] = inc - v + carry
            return carry + jnp.sum(v)
        s_t = lax.fori_loop(0, 8, lscan, jnp.int32(0))

        TSbuf[...] = jnp.full((16,), 1, jnp.int32) * s_t
        pltpu.sync_copy(TSbuf, TS2.at[pl.ds(s * 16, 16)])
        plsc.subcore_barrier()
        pltpu.sync_copy(TS2, TSl)
        svec = plsc.load_gather(TSl, [lanes * 16])
        base_t = jnp.sum(jnp.where(lanes < s, svec, 0))

        def fin(g, _):
            gb = lexcl[pl.ds(g * 16, 16)] + base_t
            def inner2(u, _):
                off = u * 128 + g * 16
                CURbuf[pl.ds(off, 16)] = CURbuf[pl.ds(off, 16)] + gb
                return 0
            lax.fori_loop(0, NT, inner2, 0)
            return 0
        lax.fori_loop(0, 8, fin, 0)

        def wr(u, _):
            pltpu.sync_copy(CURbuf.at[pl.ds(u * 128, 128)],
                            CURG.at[pl.ds(u * RADIX + s * 128, 128)])
            return 0
        lax.fori_loop(0, NT, wr, 0)
        plsc.subcore_barrier()

        pltpu.sync_copy(CURG.at[pl.ds(s * RADIX, RADIX)], cur)

        def split(g, _):
            sl = pl.ds(g * 16, 16)
            c0 = cur[sl]
            c1 = c0 + hu[0][sl]
            c2 = c1 + hu[1][sl]
            c3 = c2 + hu[2][sl]
            cu[0][sl] = c0
            cu[1][sl] = c1
            cu[2][sl] = c2
            cu[3][sl] = c3
            return 0
        lax.fori_loop(0, RADIX // 16, split, 0)

        def ssweep(i, _):
            for u in range(U):
                off = u * CU + i * 16
                d = digit(off)
                cnt, lastm = plsc.scan_count(d)
                cc = plsc.load_gather(cu[u], [d])
                posbuf[pl.ds(off, 16)] = cc + cnt - 1
                plsc.store_scatter(cu[u], [d], cc + cnt, mask=lastm)
            return 0
        lax.fori_loop(0, NVU, ssweep, 0)

        cp1 = pltpu.async_copy(keych, dst_k.at[posbuf], sem1)
        cp2 = pltpu.async_copy(valch, dst_v.at[posbuf], sem2)
        cp1.wait()
        cp2.wait()
        plsc.subcore_barrier()

    def do_row(r, _):
        row = c * ROWS_PER_CORE + r
        in_base = row * KP + s * C
        pltpu.sync_copy(scores_hbm.at[pl.ds(in_base, C)], scorech)

        def xform(i, _):
            for u in range(U):
                off = u * CU + i * 16
                b = plsc.bitcast(scorech[pl.ds(off, 16)], jnp.uint32)
                asc = jnp.where(b >= jnp.uint32(_SIGN_BITS), ~b,
                                b | jnp.uint32(_SIGN_BITS))
                keych[pl.ds(off, 16)] = plsc.bitcast(~asc, jnp.int32)
                valch[pl.ds(off, 16)] = s * C + off + lanes
            return 0
        lax.fori_loop(0, NVU, xform, 0)

        @pl.when(s == NT - 1)
        def _():
            for j in range((C - TAIL) // 16):
                keych[pl.ds(TAIL + j * 16, 16)] = zeros16 - 1
        do_pass(SHIFTS[0], KA, VA)
        pltpu.sync_copy(KA.at[pl.ds(s * C, C)], keych)
        pltpu.sync_copy(VA.at[pl.ds(s * C, C)], valch)
        do_pass(SHIFTS[1], KB, VB)
        pltpu.sync_copy(KB.at[pl.ds(s * C, C)], keych)
        pltpu.sync_copy(VB.at[pl.ds(s * C, C)], valch)
        do_pass(SHIFTS[2], KA, VA)

        pltpu.sync_copy(KA.at[pl.ds(s * C, C)], keych)
        pltpu.sync_copy(VA.at[pl.ds(s * C, C)], valch)

        def inv(i, _):
            for u in range(U):
                off = u * CU + i * 16
                kk = plsc.bitcast(keych[pl.ds(off, 16)], jnp.uint32)
                asc = ~kk
                b = jnp.where(asc >= jnp.uint32(_SIGN_BITS),
                              asc & jnp.uint32(0x7FFFFFFF), ~asc)
                scorech[pl.ds(off, 16)] = plsc.bitcast(b, jnp.float32)
            return 0
        lax.fori_loop(0, NVU, inv, 0)

        out_base = row * K + s * C

        @pl.when(s < NT - 1)
        def _():
            pltpu.sync_copy(scorech, sc_out.at[pl.ds(out_base, C)])
            pltpu.sync_copy(valch, ord_out.at[pl.ds(out_base, C)])

        @pl.when(s == NT - 1)
        def _():
            pltpu.sync_copy(scorech.at[pl.ds(0, TAIL)],
                            sc_out.at[pl.ds(out_base, TAIL)])
            pltpu.sync_copy(valch.at[pl.ds(0, TAIL)],
                            ord_out.at[pl.ds(out_base, TAIL)])

        plsc.subcore_barrier()
        return 0

    lax.fori_loop(0, ROWS_PER_CORE, do_row, 0)


@jax.jit
def kernel(queries, keys):
    scores = _scores(queries, keys)
    sorted_scores, order = _sort_kernel(scores.reshape(-1))
    return sorted_scores.reshape(Q, K), order.reshape(Q, K)


# speedup vs baseline: 1.2156x; 1.2156x over previous
import functools

import jax
import jax.numpy as jnp
from jax import lax
from jax.experimental import pallas as pl
from jax.experimental.pallas import tpu as pltpu
from jax.experimental.pallas import tpu_sc as plsc

Q = 1024
K = 100000
D = 128
EPS = 1e-8
KBLK = 2048

NT = 16
NC = 2
C = 6272
KP = NT * C
NV = C // 16
U = 4
NVU = NV // U
CU = NVU * 16
RADIX = 2048
SHIFTS = (0, 11, 22)
ROWS_PER_CORE = Q // NC
TAIL = K - (NT - 1) * C

_SIGN_BITS = 0x80000000


def _scores_body(q_ref, k_ref, qn_ref, kn_ref, out_ref):
    dot = jax.lax.dot_general(q_ref[...], k_ref[...], (((1,), (1,)), ((), ())))
    denom = jnp.maximum(qn_ref[...] * kn_ref[0:1, :], EPS)
    out_ref[...] = dot / denom


def _scores(queries, keys):
    qn = jnp.linalg.norm(queries, axis=1, keepdims=True)
    kn = jnp.linalg.norm(keys, axis=1, keepdims=True)
    kn2d = jnp.broadcast_to(kn.T, (8, K))
    return pl.pallas_call(
        _scores_body,
        grid=(pl.cdiv(K, KBLK),),
        in_specs=[
            pl.BlockSpec((Q, D), lambda i: (0, 0)),
            pl.BlockSpec((KBLK, D), lambda i: (i, 0)),
            pl.BlockSpec((Q, 1), lambda i: (0, 0)),
            pl.BlockSpec((8, KBLK), lambda i: (0, i)),
        ],
        out_specs=pl.BlockSpec((Q, KBLK), lambda i: (0, i)),
        out_shape=jax.ShapeDtypeStruct((Q, KP), jnp.float32),
    )(queries, keys, qn, kn2d)


_mesh = plsc.VectorSubcoreMesh(core_axis_name="c", subcore_axis_name="s")


@functools.partial(
    pl.kernel,
    out_type=[
        jax.ShapeDtypeStruct((Q * K,), jnp.float32),
        jax.ShapeDtypeStruct((Q * K,), jnp.int32),
    ],
    mesh=_mesh,
    compiler_params=pltpu.CompilerParams(needs_layout_passes=False),
    scratch_types=[
        pltpu.VMEM_SHARED((KP,), jnp.int32),
        pltpu.VMEM_SHARED((KP,), jnp.int32),
        pltpu.VMEM_SHARED((KP,), jnp.int32),
        pltpu.VMEM_SHARED((KP,), jnp.int32),
        pltpu.VMEM_SHARED((NT * RADIX,), jnp.int32),
        pltpu.VMEM_SHARED((NT * RADIX,), jnp.int32),
        pltpu.VMEM_SHARED((NT * 16,), jnp.int32),
        pltpu.VMEM((C,), jnp.int32),
        pltpu.VMEM((C,), jnp.int32),
        pltpu.VMEM((C,), jnp.int32),
        pltpu.VMEM((C,), jnp.int32),
        pltpu.VMEM((C,), jnp.float32),
        [pltpu.VMEM((RADIX,), jnp.int32) for _ in range(U)],
        [pltpu.VMEM((RADIX,), jnp.int32) for _ in range(U)],
        pltpu.VMEM((RADIX,), jnp.int32),
        pltpu.VMEM((RADIX,), jnp.int32),
        pltpu.VMEM((NT * 128,), jnp.int32),
        pltpu.VMEM((NT * 128,), jnp.int32),
        pltpu.VMEM((128,), jnp.int32),
        pltpu.VMEM((NT * 16,), jnp.int32),
        pltpu.VMEM((16,), jnp.int32),
        pltpu.SemaphoreType.DMA,
        pltpu.SemaphoreType.DMA,
    ],
)
def _sort_kernel(scores_hbm, sc_out, ord_out, KA, VA, KB, VB, HG, CURG, TS2,
                 keych, valch, posbuf, rankbuf, scorech, hu, cu, hist, cur, A, CURbuf,
                 lexcl, TSl, TSbuf, sem1, sem2):
    c = lax.axis_index("c")
    s = lax.axis_index("s")
    lanes = lax.iota(jnp.int32, 16)
    zeros16 = jnp.zeros((16,), jnp.int32)

    def do_pass(shift, dst_k, dst_v):
        def z(i, _):
            for u in range(U):
                hu[u][pl.ds(i * 16, 16)] = zeros16
            return 0
        lax.fori_loop(0, RADIX // 16, z, 0)

        def digit(off):
            kk = plsc.bitcast(keych[pl.ds(off, 16)], jnp.uint32)
            return ((kk >> shift) & jnp.uint32(RADIX - 1)).astype(jnp.int32)

        def hsweep(i, _):
            for u in range(U):
                off = u * CU + i * 16
                d = digit(off)
                cnt, lastm = plsc.scan_count(d)
                old = plsc.load_gather(hu[u], [d])
                rankbuf[pl.ds(off, 16)] = old + cnt - 1
                plsc.store_scatter(hu[u], [d], old + cnt, mask=lastm)
            return 0
        lax.fori_loop(0, NVU, hsweep, 0)

        def comb(g, _):
            hist[pl.ds(g * 16, 16)] = (
                hu[0][pl.ds(g * 16, 16)] + hu[1][pl.ds(g * 16, 16)]
                + hu[2][pl.ds(g * 16, 16)] + hu[3][pl.ds(g * 16, 16)])
            return 0
        lax.fori_loop(0, RADIX // 16, comb, 0)

        pltpu.sync_copy(hist, HG.at[pl.ds(s * RADIX, RADIX)])
        plsc.subcore_barrier()

        def rd(u, _):
            pltpu.sync_copy(HG.at[pl.ds(u * RADIX + s * 128, 128)],
                            A.at[pl.ds(u * 128, 128)])
            return 0
        lax.fori_loop(0, NT, rd, 0)

        def grp(g, _):
            def inner(u, acc):
                CURbuf[pl.ds(u * 128 + g * 16, 16)] = acc
                return acc + A[pl.ds(u * 128 + g * 16, 16)]
            tot = lax.fori_loop(0, NT, inner, zeros16)
            lexcl[pl.ds(g * 16, 16)] = tot
            return 0
        lax.fori_loop(0, 8, grp, 0)

        def lscan(g, carry):
            v = lexcl[pl.ds(g * 16, 16)]
            inc = plsc.cumsum(v)
            lexcl[pl.ds(g * 16, 16)] = inc - v + carry
            return carry + jnp.sum(v)
        s_t = lax.fori_loop(0, 8, lscan, jnp.int32(0))

        TSbuf[...] = jnp.full((16,), 1, jnp.int32) * s_t
        pltpu.sync_copy(TSbuf, TS2.at[pl.ds(s * 16, 16)])
        plsc.subcore_barrier()
        pltpu.sync_copy(TS2, TSl)
        svec = plsc.load_gather(TSl, [lanes * 16])
        base_t = jnp.sum(jnp.where(lanes < s, svec, 0))

        def fin(g, _):
            gb = lexcl[pl.ds(g * 16, 16)] + base_t
            def inner2(u, _):
                off = u * 128 + g * 16
                CURbuf[pl.ds(off, 16)] = CURbuf[pl.ds(off, 16)] + gb
                return 0
            lax.fori_loop(0, NT, inner2, 0)
            return 0
        lax.fori_loop(0, 8, fin, 0)

        def wr(u, _):
            pltpu.sync_copy(CURbuf.at[pl.ds(u * 128, 128)],
                            CURG.at[pl.ds(u * RADIX + s * 128, 128)])
            return 0
        lax.fori_loop(0, NT, wr, 0)
        plsc.subcore_barrier()

        pltpu.sync_copy(CURG.at[pl.ds(s * RADIX, RADIX)], cur)

        def split(g, _):
            sl = pl.ds(g * 16, 16)
            c0 = cur[sl]
            c1 = c0 + hu[0][sl]
            c2 = c1 + hu[1][sl]
            c3 = c2 + hu[2][sl]
            cu[0][sl] = c0
            cu[1][sl] = c1
            cu[2][sl] = c2
            cu[3][sl] = c3
            return 0
        lax.fori_loop(0, RADIX // 16, split, 0)

        def ssweep(i, _):
            for u in range(U):
                off = u * CU + i * 16
                d = digit(off)
                cc = plsc.load_gather(cu[u], [d])
                posbuf[pl.ds(off, 16)] = cc + rankbuf[pl.ds(off, 16)]
            return 0
        lax.fori_loop(0, NVU, ssweep, 0)

        cp1 = pltpu.async_copy(keych, dst_k.at[posbuf], sem1)
        cp2 = pltpu.async_copy(valch, dst_v.at[posbuf], sem2)
        cp1.wait()
        cp2.wait()
        plsc.subcore_barrier()

    def do_row(r, _):
        row = c * ROWS_PER_CORE + r
        in_base = row * KP + s * C
        pltpu.sync_copy(scores_hbm.at[pl.ds(in_base, C)], scorech)

        def xform(i, _):
            for u in range(U):
                off = u * CU + i * 16
                b = plsc.bitcast(scorech[pl.ds(off, 16)], jnp.uint32)
                asc = jnp.where(b >= jnp.uint32(_SIGN_BITS), ~b,
                                b | jnp.uint32(_SIGN_BITS))
                keych[pl.ds(off, 16)] = plsc.bitcast(~asc, jnp.int32)
                valch[pl.ds(off, 16)] = s * C + off + lanes
            return 0
        lax.fori_loop(0, NVU, xform, 0)

        @pl.when(s == NT - 1)
        def _():
            for j in range((C - TAIL) // 16):
                keych[pl.ds(TAIL + j * 16, 16)] = zeros16 - 1
        do_pass(SHIFTS[0], KA, VA)
        pltpu.sync_copy(KA.at[pl.ds(s * C, C)], keych)
        pltpu.sync_copy(VA.at[pl.ds(s * C, C)], valch)
        do_pass(SHIFTS[1], KB, VB)
        pltpu.sync_copy(KB.at[pl.ds(s * C, C)], keych)
        pltpu.sync_copy(VB.at[pl.ds(s * C, C)], valch)
        do_pass(SHIFTS[2], KA, VA)

        pltpu.sync_copy(KA.at[pl.ds(s * C, C)], keych)
        pltpu.sync_copy(VA.at[pl.ds(s * C, C)], valch)

        def inv(i, _):
            for u in range(U):
                off = u * CU + i * 16
                kk = plsc.bitcast(keych[pl.ds(off, 16)], jnp.uint32)
                asc = ~kk
                b = jnp.where(asc >= jnp.uint32(_SIGN_BITS),
                              asc & jnp.uint32(0x7FFFFFFF), ~asc)
                scorech[pl.ds(off, 16)] = plsc.bitcast(b, jnp.float32)
            return 0
        lax.fori_loop(0, NVU, inv, 0)

        out_base = row * K + s * C

        @pl.when(s < NT - 1)
        def _():
            pltpu.sync_copy(scorech, sc_out.at[pl.ds(out_base, C)])
            pltpu.sync_copy(valch, ord_out.at[pl.ds(out_base, C)])

        @pl.when(s == NT - 1)
        def _():
            pltpu.sync_copy(scorech.at[pl.ds(0, TAIL)],
                            sc_out.at[pl.ds(out_base, TAIL)])
            pltpu.sync_copy(valch.at[pl.ds(0, TAIL)],
                            ord_out.at[pl.ds(out_base, TAIL)])

        plsc.subcore_barrier()
        return 0

    lax.fori_loop(0, ROWS_PER_CORE, do_row, 0)


@jax.jit
def kernel(queries, keys):
    scores = _scores(queries, keys)
    sorted_scores, order = _sort_kernel(scores.reshape(-1))
    return sorted_scores.reshape(Q, K), order.reshape(Q, K)


# speedup vs baseline: 1.3753x; 1.1313x over previous
import functools

import jax
import jax.numpy as jnp
from jax import lax
from jax.experimental import pallas as pl
from jax.experimental.pallas import tpu as pltpu
from jax.experimental.pallas import tpu_sc as plsc

Q = 1024
K = 100000
D = 128
EPS = 1e-8
KBLK = 2048

NT = 16
NC = 2
C = 6272
KP = NT * C
NV = C // 16
U = 4
NVU = NV // U
CU = NVU * 16
RADIX = 2048
SHIFTS = (0, 11, 22)
ROWS_PER_CORE = Q // NC
TAIL = K - (NT - 1) * C

_SIGN_BITS = 0x80000000


def _scores_body(q_ref, k_ref, qn_ref, kn_ref, out_ref):
    dot = jax.lax.dot_general(q_ref[...], k_ref[...], (((1,), (1,)), ((), ())))
    denom = jnp.maximum(qn_ref[...] * kn_ref[0:1, :], EPS)
    out_ref[...] = dot / denom


def _scores(queries, keys):
    qn = jnp.linalg.norm(queries, axis=1, keepdims=True)
    kn = jnp.linalg.norm(keys, axis=1, keepdims=True)
    kn2d = jnp.broadcast_to(kn.T, (8, K))
    return pl.pallas_call(
        _scores_body,
        grid=(pl.cdiv(K, KBLK),),
        in_specs=[
            pl.BlockSpec((Q, D), lambda i: (0, 0)),
            pl.BlockSpec((KBLK, D), lambda i: (i, 0)),
            pl.BlockSpec((Q, 1), lambda i: (0, 0)),
            pl.BlockSpec((8, KBLK), lambda i: (0, i)),
        ],
        out_specs=pl.BlockSpec((Q, KBLK), lambda i: (0, i)),
        out_shape=jax.ShapeDtypeStruct((Q, KP), jnp.float32),
    )(queries, keys, qn, kn2d)


_mesh = plsc.VectorSubcoreMesh(core_axis_name="c", subcore_axis_name="s")


@functools.partial(
    pl.kernel,
    out_type=[
        jax.ShapeDtypeStruct((Q * K,), jnp.float32),
        jax.ShapeDtypeStruct((Q * K,), jnp.int32),
    ],
    mesh=_mesh,
    compiler_params=pltpu.CompilerParams(needs_layout_passes=False),
    scratch_types=[
        pltpu.VMEM_SHARED((KP,), jnp.int32),
        pltpu.VMEM_SHARED((KP,), jnp.int32),
        pltpu.VMEM_SHARED((KP,), jnp.int32),
        pltpu.VMEM_SHARED((KP,), jnp.int32),
        pltpu.VMEM_SHARED((NT * RADIX,), jnp.int32),
        pltpu.VMEM_SHARED((NT * RADIX,), jnp.int32),
        pltpu.VMEM_SHARED((NT * 16,), jnp.int32),
        pltpu.VMEM((C,), jnp.int32),
        pltpu.VMEM((C,), jnp.int32),
        pltpu.VMEM((C,), jnp.int32),
        pltpu.VMEM((C,), jnp.int32),
        pltpu.VMEM((C,), jnp.float32),
        [pltpu.VMEM((RADIX,), jnp.int32) for _ in range(U)],
        [pltpu.VMEM((RADIX,), jnp.int32) for _ in range(U)],
        pltpu.VMEM((RADIX,), jnp.int32),
        pltpu.VMEM((RADIX,), jnp.int32),
        pltpu.VMEM((NT * 128,), jnp.int32),
        pltpu.VMEM((NT * 128,), jnp.int32),
        pltpu.VMEM((128,), jnp.int32),
        pltpu.VMEM((NT * 16,), jnp.int32),
        pltpu.VMEM((16,), jnp.int32),
        pltpu.SemaphoreType.DMA,
        pltpu.SemaphoreType.DMA,
    ],
)
def _sort_kernel(scores_hbm, sc_out, ord_out, KA, VA, KB, VB, HG, CURG, TS2,
                 keych, valch, posbuf, rankbuf, scorech, hu, cu, hist, cur, A, CURbuf,
                 lexcl, TSl, TSbuf, sem1, sem2):
    c = lax.axis_index("c")
    s = lax.axis_index("s")
    lanes = lax.iota(jnp.int32, 16)
    zeros16 = jnp.zeros((16,), jnp.int32)

    def do_pass(shift, dst_k, dst_v):
        def z(i, _):
            for u in range(U):
                hu[u][pl.ds(i * 16, 16)] = zeros16
            return 0
        lax.fori_loop(0, RADIX // 16, z, 0)

        def digit(off):
            kk = plsc.bitcast(keych[pl.ds(off, 16)], jnp.uint32)
            return ((kk >> shift) & jnp.uint32(RADIX - 1)).astype(jnp.int32)

        def hsweep(i, _):
            for u in range(U):
                off = u * CU + i * 16
                d = digit(off)
                cnt, lastm = plsc.scan_count(d)
                old = plsc.load_gather(hu[u], [d])
                rankbuf[pl.ds(off, 16)] = old + cnt - 1
                plsc.store_scatter(hu[u], [d], old + cnt, mask=lastm)
            return 0
        lax.fori_loop(0, NVU, hsweep, 0)

        def comb(g, _):
            hist[pl.ds(g * 16, 16)] = (
                hu[0][pl.ds(g * 16, 16)] + hu[1][pl.ds(g * 16, 16)]
                + hu[2][pl.ds(g * 16, 16)] + hu[3][pl.ds(g * 16, 16)])
            return 0
        lax.fori_loop(0, RADIX // 16, comb, 0)

        pltpu.sync_copy(hist, HG.at[pl.ds(s * RADIX, RADIX)])
        plsc.subcore_barrier()

        rd_cps = [pltpu.async_copy(HG.at[pl.ds(u * RADIX + s * 128, 128)],
                                   A.at[pl.ds(u * 128, 128)], sem1)
                  for u in range(NT)]
        for cp in rd_cps:
            cp.wait()

        def grp(g, _):
            def inner(u, acc):
                CURbuf[pl.ds(u * 128 + g * 16, 16)] = acc
                return acc + A[pl.ds(u * 128 + g * 16, 16)]
            tot = lax.fori_loop(0, NT, inner, zeros16)
            lexcl[pl.ds(g * 16, 16)] = tot
            return 0
        lax.fori_loop(0, 8, grp, 0)

        def lscan(g, carry):
            v = lexcl[pl.ds(g * 16, 16)]
            inc = plsc.cumsum(v)
            lexcl[pl.ds(g * 16, 16)] = inc - v + carry
            return carry + jnp.sum(v)
        s_t = lax.fori_loop(0, 8, lscan, jnp.int32(0))

        TSbuf[...] = jnp.full((16,), 1, jnp.int32) * s_t
        pltpu.sync_copy(TSbuf, TS2.at[pl.ds(s * 16, 16)])
        plsc.subcore_barrier()
        pltpu.sync_copy(TS2, TSl)
        svec = plsc.load_gather(TSl, [lanes * 16])
        base_t = jnp.sum(jnp.where(lanes < s, svec, 0))

        def fin(g, _):
            gb = lexcl[pl.ds(g * 16, 16)] + base_t
            def inner2(u, _):
                off = u * 128 + g * 16
                CURbuf[pl.ds(off, 16)] = CURbuf[pl.ds(off, 16)] + gb
                return 0
            lax.fori_loop(0, NT, inner2, 0)
            return 0
        lax.fori_loop(0, 8, fin, 0)

        wr_cps = [pltpu.async_copy(CURbuf.at[pl.ds(u * 128, 128)],
                                   CURG.at[pl.ds(u * RADIX + s * 128, 128)],
                                   sem1)
                  for u in range(NT)]
        for cp in wr_cps:
            cp.wait()
        plsc.subcore_barrier()

        pltpu.sync_copy(CURG.at[pl.ds(s * RADIX, RADIX)], cur)

        def split(g, _):
            sl = pl.ds(g * 16, 16)
            c0 = cur[sl]
            c1 = c0 + hu[0][sl]
            c2 = c1 + hu[1][sl]
            c3 = c2 + hu[2][sl]
            cu[0][sl] = c0
            cu[1][sl] = c1
            cu[2][sl] = c2
            cu[3][sl] = c3
            return 0
        lax.fori_loop(0, RADIX // 16, split, 0)

        def ssweep(i, _):
            for u in range(U):
                off = u * CU + i * 16
                d = digit(off)
                cc = plsc.load_gather(cu[u], [d])
                posbuf[pl.ds(off, 16)] = cc + rankbuf[pl.ds(off, 16)]
            return 0
        lax.fori_loop(0, NVU, ssweep, 0)

        cp1 = pltpu.async_copy(keych, dst_k.at[posbuf], sem1)
        cp2 = pltpu.async_copy(valch, dst_v.at[posbuf], sem2)
        cp1.wait()
        cp2.wait()
        plsc.subcore_barrier()

    def do_row(r, _):
        row = c * ROWS_PER_CORE + r
        in_base = row * KP + s * C
        pltpu.sync_copy(scores_hbm.at[pl.ds(in_base, C)], scorech)

        def xform(i, _):
            for u in range(U):
                off = u * CU + i * 16
                b = plsc.bitcast(scorech[pl.ds(off, 16)], jnp.uint32)
                asc = jnp.where(b >= jnp.uint32(_SIGN_BITS), ~b,
                                b | jnp.uint32(_SIGN_BITS))
                keych[pl.ds(off, 16)] = plsc.bitcast(~asc, jnp.int32)
                valch[pl.ds(off, 16)] = s * C + off + lanes
            return 0
        lax.fori_loop(0, NVU, xform, 0)

        @pl.when(s == NT - 1)
        def _():
            for j in range((C - TAIL) // 16):
                keych[pl.ds(TAIL + j * 16, 16)] = zeros16 - 1
        def fetch(src_k, src_v):
            g1 = pltpu.async_copy(src_k.at[pl.ds(s * C, C)], keych, sem1)
            g2 = pltpu.async_copy(src_v.at[pl.ds(s * C, C)], valch, sem2)
            g1.wait()
            g2.wait()

        do_pass(SHIFTS[0], KA, VA)
        fetch(KA, VA)
        do_pass(SHIFTS[1], KB, VB)
        fetch(KB, VB)
        do_pass(SHIFTS[2], KA, VA)

        fetch(KA, VA)

        def inv(i, _):
            for u in range(U):
                off = u * CU + i * 16
                kk = plsc.bitcast(keych[pl.ds(off, 16)], jnp.uint32)
                asc = ~kk
                b = jnp.where(asc >= jnp.uint32(_SIGN_BITS),
                              asc & jnp.uint32(0x7FFFFFFF), ~asc)
                scorech[pl.ds(off, 16)] = plsc.bitcast(b, jnp.float32)
            return 0
        lax.fori_loop(0, NVU, inv, 0)

        out_base = row * K + s * C

        @pl.when(s < NT - 1)
        def _():
            o1 = pltpu.async_copy(scorech, sc_out.at[pl.ds(out_base, C)], sem1)
            o2 = pltpu.async_copy(valch, ord_out.at[pl.ds(out_base, C)], sem2)
            o1.wait()
            o2.wait()

        @pl.when(s == NT - 1)
        def _():
            o1 = pltpu.async_copy(scorech.at[pl.ds(0, TAIL)],
                                  sc_out.at[pl.ds(out_base, TAIL)], sem1)
            o2 = pltpu.async_copy(valch.at[pl.ds(0, TAIL)],
                                  ord_out.at[pl.ds(out_base, TAIL)], sem2)
            o1.wait()
            o2.wait()

        plsc.subcore_barrier()
        return 0

    lax.fori_loop(0, ROWS_PER_CORE, do_row, 0)


@jax.jit
def kernel(queries, keys):
    scores = _scores(queries, keys)
    sorted_scores, order = _sort_kernel(scores.reshape(-1))
    return sorted_scores.reshape(Q, K), order.reshape(Q, K)


# speedup vs baseline: 1.4475x; 1.0525x over previous
import functools

import jax
import jax.numpy as jnp
from jax import lax
from jax.experimental import pallas as pl
from jax.experimental.pallas import tpu as pltpu
from jax.experimental.pallas import tpu_sc as plsc

Q = 1024
K = 100000
D = 128
EPS = 1e-8
KBLK = 2048

NT = 16
NC = 2
C = 6272
KP = NT * C
NV = C // 16
U = 4
NVU = NV // U
CU = NVU * 16
RADIX = 2048
SHIFTS = (0, 11, 22)
ROWS_PER_CORE = Q // NC
TAIL = K - (NT - 1) * C

_SIGN_BITS = 0x80000000


def _scores_body(q_ref, k_ref, qn_ref, kn_ref, out_ref):
    dot = jax.lax.dot_general(q_ref[...], k_ref[...], (((1,), (1,)), ((), ())))
    denom = jnp.maximum(qn_ref[...] * kn_ref[0:1, :], EPS)
    out_ref[...] = dot / denom


def _scores(queries, keys):
    qn = jnp.linalg.norm(queries, axis=1, keepdims=True)
    kn = jnp.linalg.norm(keys, axis=1, keepdims=True)
    kn2d = jnp.broadcast_to(kn.T, (8, K))
    return pl.pallas_call(
        _scores_body,
        grid=(pl.cdiv(K, KBLK),),
        in_specs=[
            pl.BlockSpec((Q, D), lambda i: (0, 0)),
            pl.BlockSpec((KBLK, D), lambda i: (i, 0)),
            pl.BlockSpec((Q, 1), lambda i: (0, 0)),
            pl.BlockSpec((8, KBLK), lambda i: (0, i)),
        ],
        out_specs=pl.BlockSpec((Q, KBLK), lambda i: (0, i)),
        out_shape=jax.ShapeDtypeStruct((Q, KP), jnp.float32),
    )(queries, keys, qn, kn2d)


_mesh = plsc.VectorSubcoreMesh(core_axis_name="c", subcore_axis_name="s")


@functools.partial(
    pl.kernel,
    out_type=[
        jax.ShapeDtypeStruct((Q * K,), jnp.float32),
        jax.ShapeDtypeStruct((Q * K,), jnp.int32),
    ],
    mesh=_mesh,
    compiler_params=pltpu.CompilerParams(needs_layout_passes=False),
    scratch_types=[
        pltpu.VMEM_SHARED((KP,), jnp.int32),
        pltpu.VMEM_SHARED((KP,), jnp.int32),
        pltpu.VMEM_SHARED((KP,), jnp.int32),
        pltpu.VMEM_SHARED((KP,), jnp.int32),
        pltpu.VMEM_SHARED((NT * RADIX,), jnp.int32),
        pltpu.VMEM_SHARED((NT * RADIX,), jnp.int32),
        pltpu.VMEM_SHARED((NT * 16,), jnp.int32),
        pltpu.VMEM((C,), jnp.int32),
        pltpu.VMEM((C,), jnp.int32),
        pltpu.VMEM((C,), jnp.int32),
        pltpu.VMEM((C,), jnp.int32),
        pltpu.VMEM((C,), jnp.float32),
        [pltpu.VMEM((RADIX,), jnp.int32) for _ in range(U)],
        [pltpu.VMEM((RADIX,), jnp.int32) for _ in range(U)],
        pltpu.VMEM((RADIX,), jnp.int32),
        pltpu.VMEM((RADIX,), jnp.int32),
        pltpu.VMEM((NT * 128,), jnp.int32),
        pltpu.VMEM((NT * 128,), jnp.int32),
        pltpu.VMEM((128,), jnp.int32),
        pltpu.VMEM((NT * 16,), jnp.int32),
        pltpu.VMEM((16,), jnp.int32),
        pltpu.SemaphoreType.DMA,
        pltpu.SemaphoreType.DMA,
    ],
)
def _sort_kernel(scores_hbm, sc_out, ord_out, KA, VA, KB, VB, HG, CURG, TS2,
                 keych, valch, posbuf, rankbuf, scorech, hu, cu, hist, cur, A, CURbuf,
                 lexcl, TSl, TSbuf, sem1, sem2):
    c = lax.axis_index("c")
    s = lax.axis_index("s")
    lanes = lax.iota(jnp.int32, 16)
    zeros16 = jnp.zeros((16,), jnp.int32)

    def do_pass(shift, dst_k, dst_v, first=False):
        def digit(off):
            kk = plsc.bitcast(keych[pl.ds(off, 16)], jnp.uint32)
            return ((kk >> shift) & jnp.uint32(RADIX - 1)).astype(jnp.int32)

        def hsweep(i, _):
            for u in range(U):
                off = u * CU + i * 16
                if first:
                    b = plsc.bitcast(scorech[pl.ds(off, 16)], jnp.uint32)
                    asc = jnp.where(b >= jnp.uint32(_SIGN_BITS), ~b,
                                    b | jnp.uint32(_SIGN_BITS))
                    kk = ~asc
                    keych[pl.ds(off, 16)] = plsc.bitcast(kk, jnp.int32)
                    valch[pl.ds(off, 16)] = s * C + off + lanes
                    d = (kk & jnp.uint32(RADIX - 1)).astype(jnp.int32)
                else:
                    d = digit(off)
                cnt, lastm = plsc.scan_count(d)
                old = plsc.load_gather(hu[u], [d])
                rankbuf[pl.ds(off, 16)] = old + cnt - 1
                plsc.store_scatter(hu[u], [d], old + cnt, mask=lastm)
            return 0
        lax.fori_loop(0, NVU, hsweep, 0)

        def comb(g, _):
            hist[pl.ds(g * 16, 16)] = (
                hu[0][pl.ds(g * 16, 16)] + hu[1][pl.ds(g * 16, 16)]
                + hu[2][pl.ds(g * 16, 16)] + hu[3][pl.ds(g * 16, 16)])
            return 0
        lax.fori_loop(0, RADIX // 16, comb, 0)

        pltpu.sync_copy(hist, HG.at[pl.ds(s * RADIX, RADIX)])
        plsc.subcore_barrier()

        rd_cps = [pltpu.async_copy(HG.at[pl.ds(u * RADIX + s * 128, 128)],
                                   A.at[pl.ds(u * 128, 128)], sem1)
                  for u in range(NT)]
        for cp in rd_cps:
            cp.wait()

        def grp(g, _):
            def inner(u, acc):
                CURbuf[pl.ds(u * 128 + g * 16, 16)] = acc
                return acc + A[pl.ds(u * 128 + g * 16, 16)]
            tot = lax.fori_loop(0, NT, inner, zeros16)
            lexcl[pl.ds(g * 16, 16)] = tot
            return 0
        lax.fori_loop(0, 8, grp, 0)

        def lscan(g, carry):
            v = lexcl[pl.ds(g * 16, 16)]
            inc = plsc.cumsum(v)
            lexcl[pl.ds(g * 16, 16)] = inc - v + carry
            return carry + jnp.sum(v)
        s_t = lax.fori_loop(0, 8, lscan, jnp.int32(0))

        TSbuf[...] = jnp.full((16,), 1, jnp.int32) * s_t
        pltpu.sync_copy(TSbuf, TS2.at[pl.ds(s * 16, 16)])
        plsc.subcore_barrier()
        pltpu.sync_copy(TS2, TSl)
        svec = plsc.load_gather(TSl, [lanes * 16])
        base_t = jnp.sum(jnp.where(lanes < s, svec, 0))

        def fin(g, _):
            gb = lexcl[pl.ds(g * 16, 16)] + base_t
            def inner2(u, _):
                off = u * 128 + g * 16
                CURbuf[pl.ds(off, 16)] = CURbuf[pl.ds(off, 16)] + gb
                return 0
            lax.fori_loop(0, NT, inner2, 0)
            return 0
        lax.fori_loop(0, 8, fin, 0)

        wr_cps = [pltpu.async_copy(CURbuf.at[pl.ds(u * 128, 128)],
                                   CURG.at[pl.ds(u * RADIX + s * 128, 128)],
                                   sem1)
                  for u in range(NT)]
        for cp in wr_cps:
            cp.wait()
        plsc.subcore_barrier()

        pltpu.sync_copy(CURG.at[pl.ds(s * RADIX, RADIX)], cur)

        def split(g, _):
            sl = pl.ds(g * 16, 16)
            c0 = cur[sl]
            c1 = c0 + hu[0][sl]
            c2 = c1 + hu[1][sl]
            c3 = c2 + hu[2][sl]
            cu[0][sl] = c0
            cu[1][sl] = c1
            cu[2][sl] = c2
            cu[3][sl] = c3
            for u in range(U):
                hu[u][sl] = zeros16
            return 0
        lax.fori_loop(0, RADIX // 16, split, 0)

        def ssweep(i, _):
            for u in range(U):
                off = u * CU + i * 16
                d = digit(off)
                cc = plsc.load_gather(cu[u], [d])
                posbuf[pl.ds(off, 16)] = cc + rankbuf[pl.ds(off, 16)]
            return 0
        lax.fori_loop(0, NVU, ssweep, 0)

        cp1 = pltpu.async_copy(keych, dst_k.at[posbuf], sem1)
        cp2 = pltpu.async_copy(valch, dst_v.at[posbuf], sem2)
        cp1.wait()
        cp2.wait()
        plsc.subcore_barrier()

    def z0(i, _):
        for u in range(U):
            hu[u][pl.ds(i * 16, 16)] = zeros16
        return 0
    lax.fori_loop(0, RADIX // 16, z0, 0)

    def do_row(r, _):
        row = c * ROWS_PER_CORE + r
        in_base = row * KP + s * C
        pltpu.sync_copy(scores_hbm.at[pl.ds(in_base, C)], scorech)

        @pl.when(s == NT - 1)
        def _():
            for j in range((C - TAIL) // 16):
                scorech[pl.ds(TAIL + j * 16, 16)] = plsc.bitcast(
                    zeros16 - 1, jnp.float32)

        def fetch(src_k, src_v):
            g1 = pltpu.async_copy(src_k.at[pl.ds(s * C, C)], keych, sem1)
            g2 = pltpu.async_copy(src_v.at[pl.ds(s * C, C)], valch, sem2)
            g1.wait()
            g2.wait()

        do_pass(SHIFTS[0], KA, VA, first=True)
        fetch(KA, VA)
        do_pass(SHIFTS[1], KB, VB)
        fetch(KB, VB)
        do_pass(SHIFTS[2], KA, VA)

        fetch(KA, VA)

        def inv(i, _):
            for u in range(U):
                off = u * CU + i * 16
                kk = plsc.bitcast(keych[pl.ds(off, 16)], jnp.uint32)
                asc = ~kk
                b = jnp.where(asc >= jnp.uint32(_SIGN_BITS),
                              asc & jnp.uint32(0x7FFFFFFF), ~asc)
                scorech[pl.ds(off, 16)] = plsc.bitcast(b, jnp.float32)
            return 0
        lax.fori_loop(0, NVU, inv, 0)

        out_base = row * K + s * C

        @pl.when(s < NT - 1)
        def _():
            o1 = pltpu.async_copy(scorech, sc_out.at[pl.ds(out_base, C)], sem1)
            o2 = pltpu.async_copy(valch, ord_out.at[pl.ds(out_base, C)], sem2)
            o1.wait()
            o2.wait()

        @pl.when(s == NT - 1)
        def _():
            o1 = pltpu.async_copy(scorech.at[pl.ds(0, TAIL)],
                                  sc_out.at[pl.ds(out_base, TAIL)], sem1)
            o2 = pltpu.async_copy(valch.at[pl.ds(0, TAIL)],
                                  ord_out.at[pl.ds(out_base, TAIL)], sem2)
            o1.wait()
            o2.wait()

        plsc.subcore_barrier()
        return 0

    lax.fori_loop(0, ROWS_PER_CORE, do_row, 0)


@jax.jit
def kernel(queries, keys):
    scores = _scores(queries, keys)
    sorted_scores, order = _sort_kernel(scores.reshape(-1))
    return sorted_scores.reshape(Q, K), order.reshape(Q, K)
